# baseline (speedup 1.0000x reference)
import functools

import jax
import jax.numpy as jnp
from jax import lax
from jax.experimental import pallas as pl
from jax.experimental.pallas import tpu as pltpu
from jax.experimental.pallas import tpu_sc as plsc

VOCAB = 250002
D_MODEL = 768
NUM_CLASSES = 2

_NC, _NS = 2, 16
_NW = _NC * _NS

_ROWS = 4096
_VPAD = 253952
_V0 = 167936
_VTAIL0 = _VPAD - _ROWS
_SC_ROWS = _VTAIL0 - _V0
_RPT = _SC_ROWS // _NW
_SCCH = 64
_NCH = _RPT // _SCCH


def _proj_body(emb_ref, w_ref, b_ref, p0_ref, p1_ref):
    acc = lax.dot_general(
        w_ref[...], emb_ref[...],
        dimension_numbers=(((1,), (1,)), ((), ())),
        preferred_element_type=jnp.float32,
    ) + b_ref[...]
    p0_ref[...] = acc[0]
    p1_ref[...] = acc[1]


def _project_table_tc(emb_table, fc_w, fc_b):
    w_pad = jnp.zeros((8, D_MODEL), jnp.float32).at[:NUM_CLASSES].set(fc_w)
    b_pad = jnp.zeros((8, 1), jnp.float32).at[:NUM_CLASSES, 0].set(fc_b)
    nb = _V0 // _ROWS + 1

    def vmap(i):
        return (jnp.where(i < nb - 1, i, _VPAD // _ROWS - 1),)

    return pl.pallas_call(
        _proj_body,
        grid=(nb,),
        in_specs=[
            pl.BlockSpec((_ROWS, D_MODEL), lambda i: vmap(i) + (0,)),
            pl.BlockSpec((8, D_MODEL), lambda i: (0, 0)),
            pl.BlockSpec((8, 1), lambda i: (0, 0)),
        ],
        out_specs=[
            pl.BlockSpec((_ROWS,), vmap),
            pl.BlockSpec((_ROWS,), vmap),
        ],
        out_shape=[
            jax.ShapeDtypeStruct((_VPAD,), jnp.float32),
            jax.ShapeDtypeStruct((_VPAD,), jnp.float32),
        ],
    )(emb_table, w_pad, b_pad)


def _sc_compiler_params():
    import dataclasses
    cp = pltpu.CompilerParams(use_tc_tiling_on_sc=False)
    if "needs_layout_passes" in pltpu.CompilerParams.__dataclass_fields__:
        cp = dataclasses.replace(cp, needs_layout_passes=False)
    return cp


def _make_sc_proj():
    mesh = plsc.VectorSubcoreMesh(core_axis_name="c", subcore_axis_name="s")

    @functools.partial(
        pl.kernel,
        mesh=mesh,
        out_type=[
            jax.ShapeDtypeStruct((_SC_ROWS,), jnp.float32),
            jax.ShapeDtypeStruct((_SC_ROWS,), jnp.float32),
        ],
        scratch_types=[
            pltpu.VMEM((2, _SCCH, D_MODEL), jnp.float32),
            pltpu.VMEM((D_MODEL,), jnp.float32),
            pltpu.VMEM((D_MODEL,), jnp.float32),
            pltpu.VMEM((128,), jnp.float32),
            pltpu.VMEM((_RPT,), jnp.float32),
            pltpu.VMEM((_RPT,), jnp.float32),
            pltpu.SemaphoreType.DMA,
            pltpu.SemaphoreType.DMA,
            pltpu.SemaphoreType.DMA,
        ],
        compiler_params=_sc_compiler_params(),
    )
    def sc_proj_k(emb_hbm, w0_hbm, w1_hbm, bb_hbm, p0_hbm, p1_hbm,
                  bufs, w0_v, w1_v, bb_v, out0_v, out1_v, semA, semB, semw):
        wid = lax.axis_index("s") * _NC + lax.axis_index("c")
        t0 = _V0 + wid * _RPT
        cw0 = pltpu.async_copy(w0_hbm, w0_v, semw)
        cw1 = pltpu.async_copy(w1_hbm, w1_v, semw)
        cwb = pltpu.async_copy(bb_hbm, bb_v, semw)
        cw0.wait()
        cw1.wait()
        cwb.wait()
        lane = lax.iota(jnp.int32, 16)

        def start(slot, chunk):
            pltpu.async_copy(
                emb_hbm.at[pl.ds(t0 + chunk * _SCCH, _SCCH), :],
                bufs.at[slot],
                semA if slot == 0 else semB,
            )

        def wait_slot(slot):
            pltpu.make_async_copy(
                emb_hbm.at[pl.ds(t0, _SCCH), :],
                bufs.at[slot],
                semA if slot == 0 else semB,
            ).wait()

        start(0, 0)
        start(1, 1)

        def compute(slot, chunk):
            bv = bb_v[pl.ds(0, 16)]
            zf = jnp.zeros((16,), jnp.float32)
            for g in range(_SCCH // 16):
                rows = g * 16 + lane

                def dstep(j, accs):
                    wv0 = w0_v[pl.ds(j * 16, 16)]
                    wv1 = w1_v[pl.ds(j * 16, 16)]
                    acc = list(accs)
                    for k in range(16):
                        cols = jnp.zeros((16,), jnp.int32) + (j * 16 + k)
                        v = plsc.load_gather(bufs.at[slot], [rows, cols])
                        s = k % 4
                        acc[s] = acc[s] + v * wv0[k]
                        acc[4 + s] = acc[4 + s] + v * wv1[k]
                    return tuple(acc)

                accs = lax.fori_loop(0, D_MODEL // 16, dstep, (zf,) * 8)
                a0 = (accs[0] + accs[1]) + (accs[2] + accs[3]) + bv[0]
                a1 = (accs[4] + accs[5]) + (accs[6] + accs[7]) + bv[1]
                off = chunk * _SCCH + g * 16
                out0_v[pl.ds(off, 16)] = a0
                out1_v[pl.ds(off, 16)] = a1

        def body2(i, carry):
            c = 2 * i
            wait_slot(0)
            compute(0, c)

            @pl.when(c + 2 < _NCH)
            def _():
                start(0, c + 2)

            wait_slot(1)
            compute(1, c + 1)

            @pl.when(c + 3 < _NCH)
            def _():
                start(1, c + 3)

            return carry

        lax.fori_loop(0, _NCH // 2, body2, 0)
        pltpu.sync_copy(out0_v, p0_hbm.at[pl.ds(wid * _RPT, _RPT)])
        pltpu.sync_copy(out1_v, p1_hbm.at[pl.ds(wid * _RPT, _RPT)])

    return sc_proj_k


def _make_gather(b_per_w):
    mesh = plsc.VectorSubcoreMesh(core_axis_name="c", subcore_axis_name="s")

    @functools.partial(
        pl.kernel,
        mesh=mesh,
        out_type=[
            jax.ShapeDtypeStruct((_NW * b_per_w,), jnp.float32),
            jax.ShapeDtypeStruct((_NW * b_per_w,), jnp.float32),
        ],
        scratch_types=[
            pltpu.VMEM((b_per_w,), jnp.int32),
            pltpu.VMEM((b_per_w,), jnp.float32),
            pltpu.VMEM((b_per_w,), jnp.float32),
            pltpu.SemaphoreType.DMA,
            pltpu.SemaphoreType.DMA,
        ],
        compiler_params=pltpu.CompilerParams(use_tc_tiling_on_sc=False),
    )
    def gather_k(p0_hbm, p1_hbm, idx_hbm, out0_hbm, out1_hbm,
                 idx_v, rows0_v, rows1_v, sem0, sem1):
        wid = lax.axis_index("s") * _NC + lax.axis_index("c")
        base = wid * b_per_w
        pltpu.sync_copy(idx_hbm.at[pl.ds(base, b_per_w)], idx_v)
        c0 = pltpu.async_copy(p0_hbm.at[idx_v], rows0_v, sem0)
        c1 = pltpu.async_copy(p1_hbm.at[idx_v], rows1_v, sem1)
        c0.wait()
        c1.wait()
        pltpu.sync_copy(rows0_v, out0_hbm.at[pl.ds(base, b_per_w)])
        pltpu.sync_copy(rows1_v, out1_hbm.at[pl.ds(base, b_per_w)])

    return gather_k


def kernel(x, emb_table, fc_w, fc_b):
    B, L = x.shape
    n_tok = B * L
    b_per_w = n_tok // _NW
    bb = jnp.zeros((128,), jnp.float32).at[0].set(fc_b[0]).at[1].set(fc_b[1])
    p0t, p1t = _project_table_tc(emb_table, fc_w, fc_b)
    p0s, p1s = _make_sc_proj()(emb_table, fc_w[0], fc_w[1], bb)
    p0 = jnp.concatenate([p0t[:_V0], p0s, p0t[_VTAIL0:]])
    p1 = jnp.concatenate([p1t[:_V0], p1s, p1t[_VTAIL0:]])
    idx = x.astype(jnp.int32).reshape(n_tok)
    out0, out1 = _make_gather(b_per_w)(p0, p1, idx)
    return jnp.stack([out0, out1], axis=-1).reshape(B, L, NUM_CLASSES)


# speedup vs baseline: 1.6145x; 1.6145x over previous
import functools

import jax
import jax.numpy as jnp
from jax import lax
from jax.experimental import pallas as pl
from jax.experimental.pallas import tpu as pltpu
from jax.experimental.pallas import tpu_sc as plsc

VOCAB = 250002
D_MODEL = 768
NUM_CLASSES = 2

_NC, _NS = 2, 16
_NW = _NC * _NS

_ROWS = 4096
_VPAD = 253952
_V0 = 167936
_VTAIL0 = _VPAD - _ROWS
_SC_ROWS = _VTAIL0 - _V0
_RPT = _SC_ROWS // _NW
_SCCH = 64
_NCH = _RPT // _SCCH


def _proj_body(emb_ref, w_ref, b_ref, p0_ref, p1_ref):
    acc = lax.dot_general(
        w_ref[...], emb_ref[...],
        dimension_numbers=(((1,), (1,)), ((), ())),
        preferred_element_type=jnp.float32,
    ) + b_ref[...]
    p0_ref[...] = acc[0]
    p1_ref[...] = acc[1]


def _project_table_tc(emb_table, fc_w, fc_b):
    w_pad = jnp.zeros((8, D_MODEL), jnp.float32).at[:NUM_CLASSES].set(fc_w)
    b_pad = jnp.zeros((8, 1), jnp.float32).at[:NUM_CLASSES, 0].set(fc_b)
    nb = _V0 // _ROWS + 1

    def vmap(i):
        return (jnp.where(i < nb - 1, i, _VPAD // _ROWS - 1),)

    return pl.pallas_call(
        _proj_body,
        grid=(nb,),
        in_specs=[
            pl.BlockSpec((_ROWS, D_MODEL), lambda i: vmap(i) + (0,)),
            pl.BlockSpec((8, D_MODEL), lambda i: (0, 0)),
            pl.BlockSpec((8, 1), lambda i: (0, 0)),
        ],
        out_specs=[
            pl.BlockSpec((_ROWS,), vmap),
            pl.BlockSpec((_ROWS,), vmap),
        ],
        out_shape=[
            jax.ShapeDtypeStruct((_VPAD,), jnp.float32),
            jax.ShapeDtypeStruct((_VPAD,), jnp.float32),
        ],
    )(emb_table, w_pad, b_pad)


def _sc_compiler_params():
    import dataclasses
    cp = pltpu.CompilerParams(use_tc_tiling_on_sc=False)
    if "needs_layout_passes" in pltpu.CompilerParams.__dataclass_fields__:
        cp = dataclasses.replace(cp, needs_layout_passes=False)
    return cp


def _make_sc_proj():
    mesh = plsc.VectorSubcoreMesh(core_axis_name="c", subcore_axis_name="s")

    @functools.partial(
        pl.kernel,
        mesh=mesh,
        out_type=[
            jax.ShapeDtypeStruct((_SC_ROWS,), jnp.float32),
            jax.ShapeDtypeStruct((_SC_ROWS,), jnp.float32),
        ],
        scratch_types=[
            pltpu.VMEM((2, _SCCH, D_MODEL), jnp.float32),
            pltpu.VMEM((D_MODEL,), jnp.float32),
            pltpu.VMEM((D_MODEL,), jnp.float32),
            pltpu.VMEM((128,), jnp.float32),
            pltpu.VMEM((_RPT,), jnp.float32),
            pltpu.VMEM((_RPT,), jnp.float32),
            pltpu.SemaphoreType.DMA,
            pltpu.SemaphoreType.DMA,
            pltpu.SemaphoreType.DMA,
        ],
        compiler_params=_sc_compiler_params(),
    )
    def sc_proj_k(emb_hbm, w0_hbm, w1_hbm, bb_hbm, p0_hbm, p1_hbm,
                  bufs, w0_v, w1_v, bb_v, out0_v, out1_v, semA, semB, semw):
        wid = lax.axis_index("s") * _NC + lax.axis_index("c")
        t0 = _V0 + wid * _RPT
        cw0 = pltpu.async_copy(w0_hbm, w0_v, semw)
        cw1 = pltpu.async_copy(w1_hbm, w1_v, semw)
        cwb = pltpu.async_copy(bb_hbm, bb_v, semw)
        cw0.wait()
        cw1.wait()
        cwb.wait()
        lane = lax.iota(jnp.int32, 16)

        def start(slot, chunk):
            pltpu.async_copy(
                emb_hbm.at[pl.ds(t0 + chunk * _SCCH, _SCCH), :],
                bufs.at[slot],
                semA if slot == 0 else semB,
            )

        def wait_slot(slot):
            pltpu.make_async_copy(
                emb_hbm.at[pl.ds(t0, _SCCH), :],
                bufs.at[slot],
                semA if slot == 0 else semB,
            ).wait()

        start(0, 0)
        start(1, 1)

        def compute(slot, chunk):
            bv = bb_v[pl.ds(0, 16)]
            zf = jnp.zeros((16,), jnp.float32)
            for sv in range(_SCCH // 16):
                a0 = zf + bv[0]
                a1 = zf + bv[1]
                for h in range(2):
                    def dchunk(j, accs):
                        wv0 = w0_v[pl.ds(j * 16, 16)]
                        wv1 = w1_v[pl.ds(j * 16, 16)]
                        acc = list(accs)
                        for r in range(8):
                            row = sv * 16 + h * 8 + r
                            v = bufs[slot, row, pl.ds(j * 16, 16)]
                            acc[r] = acc[r] + v * wv0
                            acc[8 + r] = acc[8 + r] + v * wv1
                        return tuple(acc)

                    accs = lax.fori_loop(
                        0, D_MODEL // 16, dchunk, (zf,) * 16)
                    for r in range(8):
                        m = lane == (h * 8 + r)
                        a0 = jnp.where(m, a0 + jnp.sum(accs[r]), a0)
                        a1 = jnp.where(m, a1 + jnp.sum(accs[8 + r]), a1)
                off = chunk * _SCCH + sv * 16
                out0_v[pl.ds(off, 16)] = a0
                out1_v[pl.ds(off, 16)] = a1

        def body2(i, carry):
            c = 2 * i
            wait_slot(0)
            compute(0, c)

            @pl.when(c + 2 < _NCH)
            def _():
                start(0, c + 2)

            wait_slot(1)
            compute(1, c + 1)

            @pl.when(c + 3 < _NCH)
            def _():
                start(1, c + 3)

            return carry

        lax.fori_loop(0, _NCH // 2, body2, 0)
        pltpu.sync_copy(out0_v, p0_hbm.at[pl.ds(wid * _RPT, _RPT)])
        pltpu.sync_copy(out1_v, p1_hbm.at[pl.ds(wid * _RPT, _RPT)])

    return sc_proj_k


def _make_gather(b_per_w):
    mesh = plsc.VectorSubcoreMesh(core_axis_name="c", subcore_axis_name="s")

    @functools.partial(
        pl.kernel,
        mesh=mesh,
        out_type=[
            jax.ShapeDtypeStruct((_NW * b_per_w,), jnp.float32),
            jax.ShapeDtypeStruct((_NW * b_per_w,), jnp.float32),
        ],
        scratch_types=[
            pltpu.VMEM((b_per_w,), jnp.int32),
            pltpu.VMEM((b_per_w,), jnp.float32),
            pltpu.VMEM((b_per_w,), jnp.float32),
            pltpu.SemaphoreType.DMA,
            pltpu.SemaphoreType.DMA,
        ],
        compiler_params=pltpu.CompilerParams(use_tc_tiling_on_sc=False),
    )
    def gather_k(p0_hbm, p1_hbm, idx_hbm, out0_hbm, out1_hbm,
                 idx_v, rows0_v, rows1_v, sem0, sem1):
        wid = lax.axis_index("s") * _NC + lax.axis_index("c")
        base = wid * b_per_w
        pltpu.sync_copy(idx_hbm.at[pl.ds(base, b_per_w)], idx_v)
        c0 = pltpu.async_copy(p0_hbm.at[idx_v], rows0_v, sem0)
        c1 = pltpu.async_copy(p1_hbm.at[idx_v], rows1_v, sem1)
        c0.wait()
        c1.wait()
        pltpu.sync_copy(rows0_v, out0_hbm.at[pl.ds(base, b_per_w)])
        pltpu.sync_copy(rows1_v, out1_hbm.at[pl.ds(base, b_per_w)])

    return gather_k


def kernel(x, emb_table, fc_w, fc_b):
    B, L = x.shape
    n_tok = B * L
    b_per_w = n_tok // _NW
    bb = jnp.zeros((128,), jnp.float32).at[0].set(fc_b[0]).at[1].set(fc_b[1])
    p0t, p1t = _project_table_tc(emb_table, fc_w, fc_b)
    p0s, p1s = _make_sc_proj()(emb_table, fc_w[0], fc_w[1], bb)
    p0 = jnp.concatenate([p0t[:_V0], p0s, p0t[_VTAIL0:]])
    p1 = jnp.concatenate([p1t[:_V0], p1s, p1t[_VTAIL0:]])
    idx = x.astype(jnp.int32).reshape(n_tok)
    out0, out1 = _make_gather(b_per_w)(p0, p1, idx)
    return jnp.stack([out0, out1], axis=-1).reshape(B, L, NUM_CLASSES)


# speedup vs baseline: 1.6215x; 1.0043x over previous
import functools

import jax
import jax.numpy as jnp
from jax import lax
from jax.experimental import pallas as pl
from jax.experimental.pallas import tpu as pltpu
from jax.experimental.pallas import tpu_sc as plsc

VOCAB = 250002
D_MODEL = 768
NUM_CLASSES = 2

_NC, _NS = 2, 16
_NW = _NC * _NS

_ROWS = 4096
_VPAD = 253952
_V0 = 217088
_VTAIL0 = _VPAD - _ROWS
_SC_ROWS = _VTAIL0 - _V0
_RPT = _SC_ROWS // _NW
_SCCH = 64
_NCH = _RPT // _SCCH


def _proj_body(emb_ref, w_ref, b_ref, p0_ref, p1_ref):
    acc = lax.dot_general(
        w_ref[...], emb_ref[...],
        dimension_numbers=(((1,), (1,)), ((), ())),
        preferred_element_type=jnp.float32,
    ) + b_ref[...]
    p0_ref[...] = acc[0]
    p1_ref[...] = acc[1]


def _project_table_tc(emb_table, fc_w, fc_b):
    w_pad = jnp.zeros((8, D_MODEL), jnp.float32).at[:NUM_CLASSES].set(fc_w)
    b_pad = jnp.zeros((8, 1), jnp.float32).at[:NUM_CLASSES, 0].set(fc_b)
    nb = _V0 // _ROWS + 1

    def vmap(i):
        return (jnp.where(i < nb - 1, i, _VPAD // _ROWS - 1),)

    return pl.pallas_call(
        _proj_body,
        grid=(nb,),
        in_specs=[
            pl.BlockSpec((_ROWS, D_MODEL), lambda i: vmap(i) + (0,)),
            pl.BlockSpec((8, D_MODEL), lambda i: (0, 0)),
            pl.BlockSpec((8, 1), lambda i: (0, 0)),
        ],
        out_specs=[
            pl.BlockSpec((_ROWS,), vmap),
            pl.BlockSpec((_ROWS,), vmap),
        ],
        out_shape=[
            jax.ShapeDtypeStruct((_VPAD,), jnp.float32),
            jax.ShapeDtypeStruct((_VPAD,), jnp.float32),
        ],
    )(emb_table, w_pad, b_pad)


def _sc_compiler_params():
    import dataclasses
    cp = pltpu.CompilerParams(use_tc_tiling_on_sc=False)
    if "needs_layout_passes" in pltpu.CompilerParams.__dataclass_fields__:
        cp = dataclasses.replace(cp, needs_layout_passes=False)
    return cp


def _make_sc_proj():
    mesh = plsc.VectorSubcoreMesh(core_axis_name="c", subcore_axis_name="s")

    @functools.partial(
        pl.kernel,
        mesh=mesh,
        out_type=[
            jax.ShapeDtypeStruct((_SC_ROWS,), jnp.float32),
            jax.ShapeDtypeStruct((_SC_ROWS,), jnp.float32),
        ],
        scratch_types=[
            pltpu.VMEM((2, _SCCH, D_MODEL), jnp.float32),
            pltpu.VMEM((D_MODEL,), jnp.float32),
            pltpu.VMEM((D_MODEL,), jnp.float32),
            pltpu.VMEM((128,), jnp.float32),
            pltpu.VMEM((_RPT,), jnp.float32),
            pltpu.VMEM((_RPT,), jnp.float32),
            pltpu.SemaphoreType.DMA,
            pltpu.SemaphoreType.DMA,
            pltpu.SemaphoreType.DMA,
        ],
        compiler_params=_sc_compiler_params(),
    )
    def sc_proj_k(emb_hbm, w0_hbm, w1_hbm, bb_hbm, p0_hbm, p1_hbm,
                  bufs, w0_v, w1_v, bb_v, out0_v, out1_v, semA, semB, semw):
        wid = lax.axis_index("s") * _NC + lax.axis_index("c")
        t0 = _V0 + wid * _RPT
        cw0 = pltpu.async_copy(w0_hbm, w0_v, semw)
        cw1 = pltpu.async_copy(w1_hbm, w1_v, semw)
        cwb = pltpu.async_copy(bb_hbm, bb_v, semw)
        cw0.wait()
        cw1.wait()
        cwb.wait()
        lane = lax.iota(jnp.int32, 16)

        def start(slot, chunk):
            pltpu.async_copy(
                emb_hbm.at[pl.ds(t0 + chunk * _SCCH, _SCCH), :],
                bufs.at[slot],
                semA if slot == 0 else semB,
            )

        def wait_slot(slot):
            pltpu.make_async_copy(
                emb_hbm.at[pl.ds(t0, _SCCH), :],
                bufs.at[slot],
                semA if slot == 0 else semB,
            ).wait()

        start(0, 0)
        start(1, 1)

        def compute(slot, chunk):
            bv = bb_v[pl.ds(0, 16)]
            zf = jnp.zeros((16,), jnp.float32)
            for sv in range(_SCCH // 16):
                a0 = zf + bv[0]
                a1 = zf + bv[1]
                for h in range(2):
                    def dchunk(j, accs):
                        wv0 = w0_v[pl.ds(j * 16, 16)]
                        wv1 = w1_v[pl.ds(j * 16, 16)]
                        acc = list(accs)
                        for r in range(8):
                            row = sv * 16 + h * 8 + r
                            v = bufs[slot, row, pl.ds(j * 16, 16)]
                            acc[r] = acc[r] + v * wv0
                            acc[8 + r] = acc[8 + r] + v * wv1
                        return tuple(acc)

                    accs = lax.fori_loop(
                        0, D_MODEL // 16, dchunk, (zf,) * 16)
                    for r in range(8):
                        m = lane == (h * 8 + r)
                        a0 = jnp.where(m, a0 + jnp.sum(accs[r]), a0)
                        a1 = jnp.where(m, a1 + jnp.sum(accs[8 + r]), a1)
                off = chunk * _SCCH + sv * 16
                out0_v[pl.ds(off, 16)] = a0
                out1_v[pl.ds(off, 16)] = a1

        def body2(i, carry):
            c = 2 * i
            wait_slot(0)
            compute(0, c)

            @pl.when(c + 2 < _NCH)
            def _():
                start(0, c + 2)

            wait_slot(1)
            compute(1, c + 1)

            @pl.when(c + 3 < _NCH)
            def _():
                start(1, c + 3)

            return carry

        lax.fori_loop(0, _NCH // 2, body2, 0)
        pltpu.sync_copy(out0_v, p0_hbm.at[pl.ds(wid * _RPT, _RPT)])
        pltpu.sync_copy(out1_v, p1_hbm.at[pl.ds(wid * _RPT, _RPT)])

    return sc_proj_k


def _make_gather(b_per_w):
    mesh = plsc.VectorSubcoreMesh(core_axis_name="c", subcore_axis_name="s")

    @functools.partial(
        pl.kernel,
        mesh=mesh,
        out_type=[
            jax.ShapeDtypeStruct((_NW * b_per_w,), jnp.float32),
            jax.ShapeDtypeStruct((_NW * b_per_w,), jnp.float32),
        ],
        scratch_types=[
            pltpu.VMEM((b_per_w,), jnp.int32),
            pltpu.VMEM((b_per_w,), jnp.float32),
            pltpu.VMEM((b_per_w,), jnp.float32),
            pltpu.SemaphoreType.DMA,
            pltpu.SemaphoreType.DMA,
        ],
        compiler_params=pltpu.CompilerParams(use_tc_tiling_on_sc=False),
    )
    def gather_k(p0_hbm, p1_hbm, idx_hbm, out0_hbm, out1_hbm,
                 idx_v, rows0_v, rows1_v, sem0, sem1):
        wid = lax.axis_index("s") * _NC + lax.axis_index("c")
        base = wid * b_per_w
        pltpu.sync_copy(idx_hbm.at[pl.ds(base, b_per_w)], idx_v)
        c0 = pltpu.async_copy(p0_hbm.at[idx_v], rows0_v, sem0)
        c1 = pltpu.async_copy(p1_hbm.at[idx_v], rows1_v, sem1)
        c0.wait()
        c1.wait()
        pltpu.sync_copy(rows0_v, out0_hbm.at[pl.ds(base, b_per_w)])
        pltpu.sync_copy(rows1_v, out1_hbm.at[pl.ds(base, b_per_w)])

    return gather_k


def kernel(x, emb_table, fc_w, fc_b):
    B, L = x.shape
    n_tok = B * L
    b_per_w = n_tok // _NW
    bb = jnp.zeros((128,), jnp.float32).at[0].set(fc_b[0]).at[1].set(fc_b[1])
    p0t, p1t = _project_table_tc(emb_table, fc_w, fc_b)
    p0s, p1s = _make_sc_proj()(emb_table, fc_w[0], fc_w[1], bb)
    p0 = jnp.concatenate([p0t[:_V0], p0s, p0t[_VTAIL0:]])
    p1 = jnp.concatenate([p1t[:_V0], p1s, p1t[_VTAIL0:]])
    idx = x.astype(jnp.int32).reshape(n_tok)
    out0, out1 = _make_gather(b_per_w)(p0, p1, idx)
    return jnp.stack([out0, out1], axis=-1).reshape(B, L, NUM_CLASSES)


# speedup vs baseline: 5.0574x; 3.1190x over previous
import functools

import jax
import jax.numpy as jnp
from jax import lax
from jax.experimental import pallas as pl
from jax.experimental.pallas import tpu as pltpu
from jax.experimental.pallas import tpu_sc as plsc

VOCAB = 250002
D_MODEL = 768
NUM_CLASSES = 2


_ROWS = 4096
_VPAD = 253952


def _proj_body(emb_ref, w_ref, b_ref, p0_ref, p1_ref):
    acc = lax.dot_general(
        w_ref[...], emb_ref[...],
        dimension_numbers=(((1,), (1,)), ((), ())),
        preferred_element_type=jnp.float32,
    ) + b_ref[...]
    p0_ref[...] = acc[0]
    p1_ref[...] = acc[1]


def _project_table(emb_table, fc_w, fc_b):
    w_pad = jnp.zeros((8, D_MODEL), jnp.float32).at[:NUM_CLASSES].set(fc_w)
    b_pad = jnp.zeros((8, 1), jnp.float32).at[:NUM_CLASSES, 0].set(fc_b)
    nb = _VPAD // _ROWS
    return pl.pallas_call(
        _proj_body,
        grid=(nb,),
        in_specs=[
            pl.BlockSpec((_ROWS, D_MODEL), lambda i: (i, 0)),
            pl.BlockSpec((8, D_MODEL), lambda i: (0, 0)),
            pl.BlockSpec((8, 1), lambda i: (0, 0)),
        ],
        out_specs=[
            pl.BlockSpec((_ROWS,), lambda i: (i,)),
            pl.BlockSpec((_ROWS,), lambda i: (i,)),
        ],
        out_shape=[
            jax.ShapeDtypeStruct((_VPAD,), jnp.float32),
            jax.ShapeDtypeStruct((_VPAD,), jnp.float32),
        ],
    )(emb_table, w_pad, b_pad)


_NC, _NS = 2, 16
_NW = _NC * _NS


def _make_gather(b_per_w):
    mesh = plsc.VectorSubcoreMesh(core_axis_name="c", subcore_axis_name="s")

    @functools.partial(
        pl.kernel,
        mesh=mesh,
        out_type=[
            jax.ShapeDtypeStruct((_NW * b_per_w,), jnp.float32),
            jax.ShapeDtypeStruct((_NW * b_per_w,), jnp.float32),
        ],
        scratch_types=[
            pltpu.VMEM((b_per_w,), jnp.int32),
            pltpu.VMEM((b_per_w,), jnp.float32),
            pltpu.VMEM((b_per_w,), jnp.float32),
            pltpu.SemaphoreType.DMA,
            pltpu.SemaphoreType.DMA,
        ],
        compiler_params=pltpu.CompilerParams(use_tc_tiling_on_sc=False),
    )
    def gather_k(p0_hbm, p1_hbm, idx_hbm, out0_hbm, out1_hbm,
                 idx_v, rows0_v, rows1_v, sem0, sem1):
        wid = lax.axis_index("s") * _NC + lax.axis_index("c")
        base = wid * b_per_w
        pltpu.sync_copy(idx_hbm.at[pl.ds(base, b_per_w)], idx_v)
        c0 = pltpu.async_copy(p0_hbm.at[idx_v], rows0_v, sem0)
        c1 = pltpu.async_copy(p1_hbm.at[idx_v], rows1_v, sem1)
        c0.wait()
        c1.wait()
        pltpu.sync_copy(rows0_v, out0_hbm.at[pl.ds(base, b_per_w)])
        pltpu.sync_copy(rows1_v, out1_hbm.at[pl.ds(base, b_per_w)])

    return gather_k


def kernel(x, emb_table, fc_w, fc_b):
    B, L = x.shape
    n_tok = B * L
    b_per_w = n_tok // _NW
    p0, p1 = _project_table(emb_table, fc_w, fc_b)
    idx = x.astype(jnp.int32).reshape(n_tok)
    out0, out1 = _make_gather(b_per_w)(p0, p1, idx)
    return jnp.stack([out0, out1], axis=-1).reshape(B, L, NUM_CLASSES)
